# Initial kernel scaffold; baseline (speedup 1.0000x reference)
import jax
import jax.numpy as jnp
from jax import lax
from jax.experimental import pallas as pl
from jax.experimental.pallas import tpu as pltpu, tpu_sc as plsc

NUM_WORKERS = 32
L = 16
V = 1000
D = 32
C = 20

B, W = 4096, 50
N_WORDS = B * W
WORDS_PER_TILE = N_WORDS // NUM_WORKERS
CHUNK_WORDS = 640
NUM_CHUNKS = WORDS_PER_TILE // CHUNK_WORDS
CHUNK_IDS = CHUNK_WORDS * C
GROUPS = CHUNK_WORDS // L


def _sc_body(ids_hbm, table_hbm, out_hbm, table_v, ids_v, out_v):
    wid = lax.axis_index("s") * 2 + lax.axis_index("c")
    word_base = wid * WORDS_PER_TILE

    pltpu.sync_copy(table_hbm, table_v)
    zeros = jnp.zeros((L,), jnp.float32)
    table_v[pl.ds(0, L)] = zeros
    table_v[pl.ds(L, L)] = zeros

    lanes = lax.iota(jnp.int32, L)

    def chunk_body(g, carry):
        chunk_word0 = word_base + g * CHUNK_WORDS
        pltpu.sync_copy(ids_hbm.at[pl.ds(chunk_word0 * C, CHUNK_IDS)], ids_v)

        def group_body(gi, carry2):
            w0 = gi * L
            id_base = (w0 + lanes) * C
            idvs = [plsc.load_gather(ids_v, [id_base + c]) * D for c in range(C)]
            out_base = (w0 + lanes) * D

            def col_body(d, carry3):
                acc = plsc.load_gather(table_v, [idvs[0] + d])
                for c in range(1, C):
                    acc = acc + plsc.load_gather(table_v, [idvs[c] + d])
                plsc.store_scatter(out_v, [out_base + d], acc)
                return carry3

            lax.fori_loop(0, D, col_body, 0)
            return carry2

        lax.fori_loop(0, GROUPS, group_body, 0)
        pltpu.sync_copy(out_v, out_hbm.at[pl.ds(chunk_word0 * D, CHUNK_WORDS * D)])
        return carry

    lax.fori_loop(0, NUM_CHUNKS, chunk_body, 0)


@jax.jit
def kernel(token_ids, table):
    ids_flat = token_ids.astype(jnp.int32).reshape(-1)
    sc_call = pl.kernel(
        _sc_body,
        out_type=jax.ShapeDtypeStruct((N_WORDS * D,), jnp.float32),
        mesh=plsc.VectorSubcoreMesh(core_axis_name="c", subcore_axis_name="s"),
        scratch_types=[
            pltpu.VMEM((V * D,), jnp.float32),
            pltpu.VMEM((CHUNK_IDS,), jnp.int32),
            pltpu.VMEM((CHUNK_WORDS * D,), jnp.float32),
        ],
    )
    out = sc_call(ids_flat, table.reshape(-1))
    return out.reshape(B, W, D)


# speedup vs baseline: 10.1330x; 10.1330x over previous
import jax
import jax.numpy as jnp
from jax import lax
from jax.experimental import pallas as pl
from jax.experimental.pallas import tpu as pltpu, tpu_sc as plsc

NUM_WORKERS = 32
L = 16
V = 1000
D = 32
C = 20

B, W = 4096, 50
N_WORDS = B * W
WORDS_PER_TILE = N_WORDS // NUM_WORKERS
CHUNK_WORDS = 640
NUM_CHUNKS = WORDS_PER_TILE // CHUNK_WORDS
CHUNK_IDS = CHUNK_WORDS * C
GROUPS = CHUNK_WORDS // L


def _sc_body(ids_hbm, table_hbm, out_hbm, table_v, ids_v, out_v):
    wid = lax.axis_index("s") * 2 + lax.axis_index("c")
    word_base = wid * WORDS_PER_TILE

    pltpu.sync_copy(table_hbm, table_v)
    zeros = jnp.zeros((L,), jnp.float32)
    table_v[pl.ds(0, L)] = zeros
    table_v[pl.ds(L, L)] = zeros

    lanes = lax.iota(jnp.int32, L)

    def chunk_body(g, carry):
        chunk_word0 = word_base + g * CHUNK_WORDS
        pltpu.sync_copy(ids_hbm.at[pl.ds(chunk_word0 * C, CHUNK_IDS)], ids_v)

        def group_body(gi, carry2):
            w0 = gi * L
            id_base = (w0 + lanes) * C
            idvs = [plsc.load_gather(ids_v, [id_base + c]) * D for c in range(C)]
            out_base = (w0 + lanes) * D

            def col_body(d, carry3):
                acc = plsc.load_gather(table_v, [idvs[0] + d])
                for c in range(1, C):
                    acc = acc + plsc.load_gather(table_v, [idvs[c] + d])
                plsc.store_scatter(out_v, [out_base + d], acc)
                return carry3

            lax.fori_loop(0, D, col_body, 0)
            return carry2

        lax.fori_loop(0, GROUPS, group_body, 0)
        pltpu.sync_copy(out_v, out_hbm.at[pl.ds(chunk_word0 * D, CHUNK_WORDS * D)])
        return carry

    lax.fori_loop(0, NUM_CHUNKS, chunk_body, 0)


@jax.jit
def kernel(token_ids, table):
    ids_flat = token_ids.astype(jnp.int32).reshape(-1)
    sc_call = pl.kernel(
        _sc_body,
        out_type=jax.ShapeDtypeStruct((N_WORDS * D,), jnp.float32),
        mesh=plsc.VectorSubcoreMesh(core_axis_name="c", subcore_axis_name="s"),
        compiler_params=pltpu.CompilerParams(needs_layout_passes=False),
        scratch_types=[
            pltpu.VMEM((V * D,), jnp.float32),
            pltpu.VMEM((CHUNK_IDS,), jnp.int32),
            pltpu.VMEM((CHUNK_WORDS * D,), jnp.float32),
        ],
    )
    out = sc_call(ids_flat, table.reshape(-1))
    return out.reshape(B, W, D)


# speedup vs baseline: 22.8183x; 2.2519x over previous
import jax
import jax.numpy as jnp
from jax import lax
from jax.experimental import pallas as pl
from jax.experimental.pallas import tpu as pltpu, tpu_sc as plsc

NUM_WORKERS = 32
L = 16
V = 1000
D = 32
DP = D + 1
C = 20

B, W = 4096, 50
N_WORDS = B * W
WORDS_PER_TILE = N_WORDS // NUM_WORKERS
CHUNK_WORDS = 640
NUM_CHUNKS = WORDS_PER_TILE // CHUNK_WORDS
CHUNK_IDS = CHUNK_WORDS * C
GROUPS = CHUNK_WORDS // L


def _sc_body(ids_hbm, table_hbm, out_hbm, table_v, ids_v, out_v):
    wid = lax.axis_index("s") * 2 + lax.axis_index("c")
    word_base = wid * WORDS_PER_TILE

    pltpu.sync_copy(table_hbm, table_v)
    zeros = jnp.zeros((L,), jnp.float32)
    table_v[0, pl.ds(0, L)] = zeros
    table_v[0, pl.ds(L, L)] = zeros

    lanes = lax.iota(jnp.int32, L)

    def chunk_body(g, carry):
        chunk_word0 = word_base + g * CHUNK_WORDS
        pltpu.sync_copy(ids_hbm.at[pl.ds(chunk_word0 * C, CHUNK_IDS)], ids_v)

        def group_body(gi, carry2):
            w0 = gi * L
            id_base = (w0 + lanes) * C
            idvs = [plsc.load_gather(ids_v, [id_base + c]) for c in range(C)]
            words = w0 + lanes

            def col_body(d, carry3):
                colv = jnp.full((L,), d, jnp.int32)
                acc = plsc.load_gather(table_v, [idvs[0], colv])
                for c in range(1, C):
                    acc = acc + plsc.load_gather(table_v, [idvs[c], colv])
                plsc.store_scatter(out_v, [words, colv], acc)
                return carry3

            lax.fori_loop(0, D, col_body, 0)
            return carry2

        lax.fori_loop(0, GROUPS, group_body, 0)
        pltpu.sync_copy(
            out_v.at[:, pl.ds(0, D)],
            out_hbm.at[pl.ds(chunk_word0, CHUNK_WORDS), :],
        )
        return carry

    lax.fori_loop(0, NUM_CHUNKS, chunk_body, 0)


@jax.jit
def kernel(token_ids, table):
    ids_flat = token_ids.astype(jnp.int32).reshape(-1)
    table_p = jnp.pad(table, ((0, 0), (0, DP - D)))
    sc_call = pl.kernel(
        _sc_body,
        out_type=jax.ShapeDtypeStruct((N_WORDS, D), jnp.float32),
        mesh=plsc.VectorSubcoreMesh(core_axis_name="c", subcore_axis_name="s"),
        compiler_params=pltpu.CompilerParams(
            needs_layout_passes=False, use_tc_tiling_on_sc=False
        ),
        scratch_types=[
            pltpu.VMEM((V, DP), jnp.float32),
            pltpu.VMEM((CHUNK_IDS,), jnp.int32),
            pltpu.VMEM((CHUNK_WORDS, DP), jnp.float32),
        ],
    )
    out = sc_call(ids_flat, table_p)
    return out.reshape(B, W, D)
